# baseline (speedup 1.0000x reference)
import functools

import jax
import jax.numpy as jnp
from jax import lax
from jax.experimental import pallas as pl
from jax.experimental.pallas import tpu as pltpu
from jax.experimental.pallas import tpu_sc as plsc

B = 4096
EMB = 64
NUM_PERIOD = 7
NT = 10001
NG = 1000
NJ = 7000
NI = 1000000
GEN = 8
PLEN = 50

OFF_TA = 0
OFF_TB = NT
OFF_G = 2 * NT
OFF_J = 2 * NT + NG
OFF_W2 = ((OFF_J + NJ + 15) // 16) * 16
OFF_BT = OFF_W2 + EMB
OFF_BS = OFF_BT + 16
OFF_BP = OFF_BS + 16
P_TOTAL = OFF_BP + 16

NC, NS = 2, 16
NW = NC * NS
BW = B // NW
LANES = 16
LINE = 128


def _proj_body(et_ref, eg_ref, ej_ref, wt_ref, ws_ref, wp_ref,
               pta_ref, ptb_ref, pg_ref, pj_ref):
    wa = wt_ref[:, 0:EMB] + wt_ref[:, 3 * EMB:4 * EMB]
    wb = wt_ref[:, 2 * EMB:3 * EMB] - wt_ref[:, 0:EMB]
    wab = jnp.concatenate([wa, wb], axis=0)
    pt = jnp.dot(wab, et_ref[:, :], **_DOT)
    pta_ref[:] = pt[0]
    ptb_ref[:] = pt[1]
    pg_ref[:] = jnp.dot(ws_ref[:, :], eg_ref[:, :], **_DOT)[0]
    pj_ref[:] = jnp.dot(wp_ref[:, :], ej_ref[:, :], **_DOT)[0]


ITEM_BLK = 16384
SC_CHUNK = 512
SC_NCHUNK = 27
SC_PER_W = SC_CHUNK * SC_NCHUNK
SCN = SC_PER_W * NW
TCN = NI - SCN
_DOT = dict(precision=jax.lax.Precision.HIGHEST,
            preferred_element_type=jnp.float32)


def _proj_item_body(et_ref, w2_ref, p_ref):
    p_ref[:] = jnp.dot(w2_ref[:, :], et_ref[:, :], **_DOT)[0]


def _project_item(embed_item_t, w2row):
    return pl.pallas_call(
        _proj_item_body,
        grid=((TCN + ITEM_BLK - 1) // ITEM_BLK,),
        in_specs=[
            pl.BlockSpec((EMB, ITEM_BLK), lambda i: (0, i + SCN // ITEM_BLK)),
            pl.BlockSpec((1, EMB), lambda i: (0, 0)),
        ],
        out_specs=pl.BlockSpec((ITEM_BLK,), lambda i: (i,)),
        out_shape=jax.ShapeDtypeStruct((TCN,), jnp.float32),
    )(embed_item_t, w2row)


@functools.partial(
    pl.kernel,
    out_type=jax.ShapeDtypeStruct((SCN,), jnp.float32),
    mesh=plsc.VectorSubcoreMesh(core_axis_name="c", subcore_axis_name="s"),
    compiler_params=pltpu.CompilerParams(
        needs_layout_passes=False, use_tc_tiling_on_sc=True),
    scratch_types=[
        pltpu.VMEM((EMB, SC_CHUNK), jnp.float32),
        pltpu.VMEM((EMB, 16), jnp.float32),
        pltpu.VMEM((SC_CHUNK,), jnp.float32),
    ],
)
def _sc_sweep(et_h, w2b_h, p_o, panel_v, w2b_v, out_v):
    wid = lax.axis_index("s") * NC + lax.axis_index("c")
    pltpu.sync_copy(w2b_h, w2b_v)

    def chunk_body(c, carry):
        cbase = wid * SC_PER_W + c * SC_CHUNK
        pltpu.sync_copy(et_h.at[:, pl.ds(cbase, SC_CHUNK)], panel_v)
        for strip in range(SC_CHUNK // 64):
            acc = [jnp.zeros((16,), jnp.float32) for _ in range(4)]
            for k in range(EMB):
                wk = w2b_v[k]
                for q in range(4):
                    col = panel_v[k, pl.ds(strip * 64 + q * 16, 16)]
                    acc[q] = acc[q] + col * wk
            for q in range(4):
                out_v[pl.ds(strip * 64 + q * 16, 16)] = acc[q]
        pltpu.sync_copy(out_v, p_o.at[pl.ds(cbase, SC_CHUNK)])
        return carry

    lax.fori_loop(0, SC_NCHUNK, chunk_body, 0)


def _project(embed_time_t, embed_genre_t, embed_joint_t,
             w_time, w_side, w_periodic):
    return pl.pallas_call(
        _proj_body,
        out_shape=[
            jax.ShapeDtypeStruct((NT,), jnp.float32),
            jax.ShapeDtypeStruct((NT,), jnp.float32),
            jax.ShapeDtypeStruct((NG,), jnp.float32),
            jax.ShapeDtypeStruct((NJ,), jnp.float32),
        ],
    )(embed_time_t, embed_genre_t, embed_joint_t, w_time, w_side, w_periodic)


@functools.partial(
    pl.kernel,
    out_type=(
        jax.ShapeDtypeStruct((B,), jnp.float32),
        jax.ShapeDtypeStruct((B,), jnp.float32),
        jax.ShapeDtypeStruct((B,), jnp.float32),
        jax.ShapeDtypeStruct((B,), jnp.float32),
    ),
    mesh=plsc.VectorSubcoreMesh(core_axis_name="c", subcore_axis_name="s"),
    compiler_params=pltpu.CompilerParams(
        needs_layout_passes=False, use_tc_tiling_on_sc=True),
    scratch_types=[
        pltpu.VMEM((BW,), jnp.int32),
        pltpu.VMEM((BW,), jnp.int32),
        pltpu.VMEM((BW,), jnp.int32),
        pltpu.VMEM((BW * GEN,), jnp.int32),
        pltpu.VMEM((BW * PLEN,), jnp.float32),
        pltpu.VMEM((BW,), jnp.int32),
        pltpu.VMEM((BW,), jnp.int32),
        pltpu.VMEM((BW,), jnp.float32),
        pltpu.VMEM((BW,), jnp.float32),
        pltpu.VMEM((P_TOTAL,), jnp.float32),
        pltpu.VMEM((BW,), jnp.float32),
        pltpu.VMEM((BW,), jnp.float32),
        pltpu.VMEM((BW,), jnp.float32),
        pltpu.VMEM((BW,), jnp.float32),
        pltpu.SemaphoreType.DMA,
    ],
)
def _sc_kernel(item_h, tr_h, t_h, genre_h, pop_h, psc_h, ptc_h, pall_h,
               pop_o, time_o, side_o, per_o,
               item_v, tr_v, t_v, genre_v, pop_v, ia_v, ib_v, sa_v, sb_v,
               pall_v, pop_b, time_b, side_b, per_b, sem):
    wid = lax.axis_index("s") * NC + lax.axis_index("c")
    base = wid * BW

    pltpu.sync_copy(item_h.at[pl.ds(base, BW)], item_v)
    pltpu.sync_copy(tr_h.at[pl.ds(base, BW)], tr_v)
    pltpu.sync_copy(t_h.at[pl.ds(base, BW)], t_v)
    pltpu.sync_copy(genre_h.at[pl.ds(base * GEN, BW * GEN)], genre_v)
    pltpu.sync_copy(pop_h.at[pl.ds(base * PLEN, BW * PLEN)], pop_v)
    pltpu.sync_copy(pall_h, pall_v)

    lanes = lax.iota(jnp.int32, 16)
    zero16 = jnp.zeros((16,), jnp.float32)
    bt = pall_v[pl.ds(OFF_BT, 16)]
    bs = pall_v[pl.ds(OFF_BS, 16)]
    bp = pall_v[pl.ds(OFF_BP, 16)]

    for g in range(BW // LANES):
        sl = pl.ds(g * LANES, LANES)
        iv = item_v[sl]
        ia_v[sl] = jnp.minimum(iv, SCN - 1)
        ib_v[sl] = jnp.maximum(iv - SCN, 0)
    da = pltpu.async_copy(psc_h.at[ia_v], sa_v, sem)
    da.wait()
    pltpu.async_copy(ptc_h.at[ib_v], sb_v, sem).wait()

    for g in range(BW // LANES):
        off = g * LANES
        sl = pl.ds(off, LANES)
        rowv = lanes + off
        s_item = jnp.where(item_v[sl] < SCN, sa_v[sl], sb_v[sl])

        tr16 = tr_v[sl]
        t16 = t_v[sl]
        s_tre = plsc.load_gather(pall_v, [tr16 + OFF_TA])
        s_te = plsc.load_gather(pall_v, [t16 + OFF_TB])
        tmod = lax.rem(t16, NUM_PERIOD)

        sg = zero16
        sj = zero16
        gbase = rowv * GEN
        for j in range(GEN):
            gv = plsc.load_gather(genre_v, [gbase + j])
            sg = sg + plsc.load_gather(pall_v, [gv + OFF_G])
            jid = (gv * NUM_PERIOD + tmod) * jnp.minimum(gv, 1)
            sj = sj + plsc.load_gather(pall_v, [jid + OFF_J])

        popc = plsc.load_gather(pop_v, [rowv * PLEN])

        pop_b[sl] = popc
        time_b[sl] = jnp.maximum(s_tre + s_te + s_item + bt, 0.0)
        side_b[sl] = jnp.maximum(sg * (1.0 / GEN) + bs, 0.0)
        per_b[sl] = jnp.maximum(sj * (1.0 / GEN) + bp, 0.0)

    pltpu.sync_copy(pop_b, pop_o.at[pl.ds(base, BW)])
    pltpu.sync_copy(time_b, time_o.at[pl.ds(base, BW)])
    pltpu.sync_copy(side_b, side_o.at[pl.ds(base, BW)])
    pltpu.sync_copy(per_b, per_o.at[pl.ds(base, BW)])


def kernel(item, time_release, item_genre, item_director, item_actor, time,
           pop_history, pop_gt, valid_pop_len,
           embed_item, embed_time, embed_genre, embed_joint,
           w_periodic, b_periodic, w_time, b_time, w_side, b_side, attn_w):
    pta, ptb, pg, pj = _project(embed_time.T, embed_genre.T, embed_joint.T,
                                w_time, w_side, w_periodic)
    w2 = w_time[:, EMB:2 * EMB]
    pitem_tc = _project_item(embed_item.T, w2)
    pitem_sc = _sc_sweep(embed_item.T, jnp.broadcast_to(w2.T, (EMB, 16)))
    pall = jnp.concatenate([
        pta, ptb, pg, pj,
        jnp.zeros((OFF_W2 - OFF_J - NJ,), jnp.float32),
        w_time[0, EMB:2 * EMB],
        jnp.full((16,), b_time[0], jnp.float32),
        jnp.full((16,), b_side[0], jnp.float32),
        jnp.full((16,), b_periodic[0], jnp.float32),
    ])
    pop_o, time_o, side_o, per_o = _sc_kernel(
        item, time_release, time,
        item_genre.reshape(-1), pop_history.reshape(-1),
        pitem_sc, pitem_tc, pall)
    w1 = attn_w[1]
    out = time_o * w1 / w1
    return (pop_o[:, None], time_o[:, None], side_o[:, None], per_o[:, None], out)


# speedup vs baseline: 1.9563x; 1.9563x over previous
import functools

import jax
import jax.numpy as jnp
from jax import lax
from jax.experimental import pallas as pl
from jax.experimental.pallas import tpu as pltpu
from jax.experimental.pallas import tpu_sc as plsc

B = 4096
EMB = 64
NUM_PERIOD = 7
NT = 10001
NG = 1000
NJ = 7000
NI = 1000000
GEN = 8
PLEN = 50

OFF_TA = 0
OFF_TB = NT
OFF_G = 2 * NT
OFF_J = 2 * NT + NG
OFF_W2 = ((OFF_J + NJ + 15) // 16) * 16
OFF_BT = OFF_W2 + EMB
OFF_BS = OFF_BT + 16
OFF_BP = OFF_BS + 16
P_TOTAL = OFF_BP + 16

NC, NS = 2, 16
NW = NC * NS
BW = B // NW
LANES = 16
LINE = 128


def _proj_body(et_ref, eg_ref, ej_ref, wt_ref, ws_ref, wp_ref,
               pta_ref, ptb_ref, pg_ref, pj_ref):
    wa = wt_ref[:, 0:EMB] + wt_ref[:, 3 * EMB:4 * EMB]
    wb = wt_ref[:, 2 * EMB:3 * EMB] - wt_ref[:, 0:EMB]
    wab = jnp.concatenate([wa, wb], axis=0)
    pt = jnp.dot(wab, et_ref[:, :], **_DOT)
    pta_ref[:] = pt[0]
    ptb_ref[:] = pt[1]
    pg_ref[:] = jnp.dot(ws_ref[:, :], eg_ref[:, :], **_DOT)[0]
    pj_ref[:] = jnp.dot(wp_ref[:, :], ej_ref[:, :], **_DOT)[0]


ITEM_BLK = 16384
SC_CHUNK = 512
SC_NCHUNK = 9
SC_PER_W = SC_CHUNK * SC_NCHUNK
SCN = SC_PER_W * NW
TCN = NI - SCN
_DOT = dict(precision=jax.lax.Precision.HIGHEST,
            preferred_element_type=jnp.float32)


def _proj_item_body(et_ref, w2_ref, p_ref):
    p_ref[:] = jnp.dot(w2_ref[:, :], et_ref[:, :], **_DOT)[0]


def _project_item(embed_item_t, w2row):
    return pl.pallas_call(
        _proj_item_body,
        grid=((TCN + ITEM_BLK - 1) // ITEM_BLK,),
        in_specs=[
            pl.BlockSpec((EMB, ITEM_BLK), lambda i: (0, i + SCN // ITEM_BLK)),
            pl.BlockSpec((1, EMB), lambda i: (0, 0)),
        ],
        out_specs=pl.BlockSpec((ITEM_BLK,), lambda i: (i,)),
        out_shape=jax.ShapeDtypeStruct((TCN,), jnp.float32),
    )(embed_item_t, w2row)


@functools.partial(
    pl.kernel,
    out_type=jax.ShapeDtypeStruct((SCN,), jnp.float32),
    mesh=plsc.VectorSubcoreMesh(core_axis_name="c", subcore_axis_name="s"),
    compiler_params=pltpu.CompilerParams(
        needs_layout_passes=False, use_tc_tiling_on_sc=True),
    scratch_types=[
        pltpu.VMEM((EMB, SC_CHUNK), jnp.float32),
        pltpu.VMEM((EMB, 16), jnp.float32),
        pltpu.VMEM((SC_CHUNK,), jnp.float32),
    ],
)
def _sc_sweep(et_h, w2b_h, p_o, panel_v, w2b_v, out_v):
    wid = lax.axis_index("s") * NC + lax.axis_index("c")
    pltpu.sync_copy(w2b_h, w2b_v)

    def chunk_body(c, carry):
        cbase = wid * SC_PER_W + c * SC_CHUNK
        pltpu.sync_copy(et_h.at[:, pl.ds(cbase, SC_CHUNK)], panel_v)
        for strip in range(SC_CHUNK // 64):
            acc = [jnp.zeros((16,), jnp.float32) for _ in range(4)]
            for k in range(EMB):
                wk = w2b_v[k]
                for q in range(4):
                    col = panel_v[k, pl.ds(strip * 64 + q * 16, 16)]
                    acc[q] = acc[q] + col * wk
            for q in range(4):
                out_v[pl.ds(strip * 64 + q * 16, 16)] = acc[q]
        pltpu.sync_copy(out_v, p_o.at[pl.ds(cbase, SC_CHUNK)])
        return carry

    lax.fori_loop(0, SC_NCHUNK, chunk_body, 0)


def _project(embed_time_t, embed_genre_t, embed_joint_t,
             w_time, w_side, w_periodic):
    return pl.pallas_call(
        _proj_body,
        out_shape=[
            jax.ShapeDtypeStruct((NT,), jnp.float32),
            jax.ShapeDtypeStruct((NT,), jnp.float32),
            jax.ShapeDtypeStruct((NG,), jnp.float32),
            jax.ShapeDtypeStruct((NJ,), jnp.float32),
        ],
    )(embed_time_t, embed_genre_t, embed_joint_t, w_time, w_side, w_periodic)


@functools.partial(
    pl.kernel,
    out_type=(
        jax.ShapeDtypeStruct((B,), jnp.float32),
        jax.ShapeDtypeStruct((B,), jnp.float32),
        jax.ShapeDtypeStruct((B,), jnp.float32),
        jax.ShapeDtypeStruct((B,), jnp.float32),
    ),
    mesh=plsc.VectorSubcoreMesh(core_axis_name="c", subcore_axis_name="s"),
    compiler_params=pltpu.CompilerParams(
        needs_layout_passes=False, use_tc_tiling_on_sc=True),
    scratch_types=[
        pltpu.VMEM((BW,), jnp.int32),
        pltpu.VMEM((BW,), jnp.int32),
        pltpu.VMEM((BW,), jnp.int32),
        pltpu.VMEM((BW * GEN,), jnp.int32),
        pltpu.VMEM((BW * PLEN,), jnp.float32),
        pltpu.VMEM((BW,), jnp.int32),
        pltpu.VMEM((BW,), jnp.int32),
        pltpu.VMEM((BW,), jnp.float32),
        pltpu.VMEM((BW,), jnp.float32),
        pltpu.VMEM((P_TOTAL,), jnp.float32),
        pltpu.VMEM((BW,), jnp.float32),
        pltpu.VMEM((BW,), jnp.float32),
        pltpu.VMEM((BW,), jnp.float32),
        pltpu.VMEM((BW,), jnp.float32),
        pltpu.SemaphoreType.DMA,
    ],
)
def _sc_kernel(item_h, tr_h, t_h, genre_h, pop_h, psc_h, ptc_h, pall_h,
               pop_o, time_o, side_o, per_o,
               item_v, tr_v, t_v, genre_v, pop_v, ia_v, ib_v, sa_v, sb_v,
               pall_v, pop_b, time_b, side_b, per_b, sem):
    wid = lax.axis_index("s") * NC + lax.axis_index("c")
    base = wid * BW

    pltpu.sync_copy(item_h.at[pl.ds(base, BW)], item_v)
    pltpu.sync_copy(tr_h.at[pl.ds(base, BW)], tr_v)
    pltpu.sync_copy(t_h.at[pl.ds(base, BW)], t_v)
    pltpu.sync_copy(genre_h.at[pl.ds(base * GEN, BW * GEN)], genre_v)
    pltpu.sync_copy(pop_h.at[pl.ds(base * PLEN, BW * PLEN)], pop_v)
    pltpu.sync_copy(pall_h, pall_v)

    lanes = lax.iota(jnp.int32, 16)
    zero16 = jnp.zeros((16,), jnp.float32)
    bt = pall_v[pl.ds(OFF_BT, 16)]
    bs = pall_v[pl.ds(OFF_BS, 16)]
    bp = pall_v[pl.ds(OFF_BP, 16)]

    for g in range(BW // LANES):
        sl = pl.ds(g * LANES, LANES)
        iv = item_v[sl]
        ia_v[sl] = jnp.minimum(iv, SCN - 1)
        ib_v[sl] = jnp.maximum(iv - SCN, 0)
    da = pltpu.async_copy(psc_h.at[ia_v], sa_v, sem)
    da.wait()
    pltpu.async_copy(ptc_h.at[ib_v], sb_v, sem).wait()

    for g in range(BW // LANES):
        off = g * LANES
        sl = pl.ds(off, LANES)
        rowv = lanes + off
        s_item = jnp.where(item_v[sl] < SCN, sa_v[sl], sb_v[sl])

        tr16 = tr_v[sl]
        t16 = t_v[sl]
        s_tre = plsc.load_gather(pall_v, [tr16 + OFF_TA])
        s_te = plsc.load_gather(pall_v, [t16 + OFF_TB])
        tmod = lax.rem(t16, NUM_PERIOD)

        sg = zero16
        sj = zero16
        gbase = rowv * GEN
        for j in range(GEN):
            gv = plsc.load_gather(genre_v, [gbase + j])
            sg = sg + plsc.load_gather(pall_v, [gv + OFF_G])
            jid = (gv * NUM_PERIOD + tmod) * jnp.minimum(gv, 1)
            sj = sj + plsc.load_gather(pall_v, [jid + OFF_J])

        popc = plsc.load_gather(pop_v, [rowv * PLEN])

        pop_b[sl] = popc
        time_b[sl] = jnp.maximum(s_tre + s_te + s_item + bt, 0.0)
        side_b[sl] = jnp.maximum(sg * (1.0 / GEN) + bs, 0.0)
        per_b[sl] = jnp.maximum(sj * (1.0 / GEN) + bp, 0.0)

    pltpu.sync_copy(pop_b, pop_o.at[pl.ds(base, BW)])
    pltpu.sync_copy(time_b, time_o.at[pl.ds(base, BW)])
    pltpu.sync_copy(side_b, side_o.at[pl.ds(base, BW)])
    pltpu.sync_copy(per_b, per_o.at[pl.ds(base, BW)])


def kernel(item, time_release, item_genre, item_director, item_actor, time,
           pop_history, pop_gt, valid_pop_len,
           embed_item, embed_time, embed_genre, embed_joint,
           w_periodic, b_periodic, w_time, b_time, w_side, b_side, attn_w):
    pta, ptb, pg, pj = _project(embed_time.T, embed_genre.T, embed_joint.T,
                                w_time, w_side, w_periodic)
    w2 = w_time[:, EMB:2 * EMB]
    pitem_tc = _project_item(embed_item.T, w2)
    pitem_sc = _sc_sweep(embed_item.T, jnp.broadcast_to(w2.T, (EMB, 16)))
    pall = jnp.concatenate([
        pta, ptb, pg, pj,
        jnp.zeros((OFF_W2 - OFF_J - NJ,), jnp.float32),
        w_time[0, EMB:2 * EMB],
        jnp.full((16,), b_time[0], jnp.float32),
        jnp.full((16,), b_side[0], jnp.float32),
        jnp.full((16,), b_periodic[0], jnp.float32),
    ])
    pop_o, time_o, side_o, per_o = _sc_kernel(
        item, time_release, time,
        item_genre.reshape(-1), pop_history.reshape(-1),
        pitem_sc, pitem_tc, pall)
    w1 = attn_w[1]
    out = time_o * w1 / w1
    return (pop_o[:, None], time_o[:, None], side_o[:, None], per_o[:, None], out)


# speedup vs baseline: 2.1942x; 1.1216x over previous
import functools

import jax
import jax.numpy as jnp
from jax import lax
from jax.experimental import pallas as pl
from jax.experimental.pallas import tpu as pltpu
from jax.experimental.pallas import tpu_sc as plsc

B = 4096
EMB = 64
NUM_PERIOD = 7
NT = 10001
NG = 1000
NJ = 7000
NI = 1000000
GEN = 8
PLEN = 50

OFF_TA = 0
OFF_TB = NT
OFF_G = 2 * NT
OFF_J = 2 * NT + NG
OFF_W2 = ((OFF_J + NJ + 15) // 16) * 16
OFF_BT = OFF_W2 + EMB
OFF_BS = OFF_BT + 16
OFF_BP = OFF_BS + 16
P_TOTAL = OFF_BP + 16

NC, NS = 2, 16
NW = NC * NS
BW = B // NW
LANES = 16
LINE = 128


def _proj_body(et_ref, eg_ref, ej_ref, wt_ref, ws_ref, wp_ref,
               pta_ref, ptb_ref, pg_ref, pj_ref):
    wa = wt_ref[:, 0:EMB] + wt_ref[:, 3 * EMB:4 * EMB]
    wb = wt_ref[:, 2 * EMB:3 * EMB] - wt_ref[:, 0:EMB]
    wab = jnp.concatenate([wa, wb], axis=0)
    pt = jnp.dot(wab, et_ref[:, :], **_DOT)
    pta_ref[:] = pt[0]
    ptb_ref[:] = pt[1]
    pg_ref[:] = jnp.dot(ws_ref[:, :], eg_ref[:, :], **_DOT)[0]
    pj_ref[:] = jnp.dot(wp_ref[:, :], ej_ref[:, :], **_DOT)[0]


ITEM_BLK = 32768
_DOT = dict(precision=jax.lax.Precision.HIGHEST,
            preferred_element_type=jnp.float32)


def _proj_item_body(et_ref, w2_ref, p_ref):
    p_ref[:] = jnp.dot(w2_ref[:, :], et_ref[:, :], **_DOT)[0]


def _project_item(embed_item_t, w2row):
    return pl.pallas_call(
        _proj_item_body,
        grid=((NI + ITEM_BLK - 1) // ITEM_BLK,),
        in_specs=[
            pl.BlockSpec((EMB, ITEM_BLK), lambda i: (0, i)),
            pl.BlockSpec((1, EMB), lambda i: (0, 0)),
        ],
        out_specs=pl.BlockSpec((ITEM_BLK,), lambda i: (i,)),
        out_shape=jax.ShapeDtypeStruct((NI,), jnp.float32),
    )(embed_item_t, w2row)


def _project(embed_time_t, embed_genre_t, embed_joint_t,
             w_time, w_side, w_periodic):
    return pl.pallas_call(
        _proj_body,
        out_shape=[
            jax.ShapeDtypeStruct((NT,), jnp.float32),
            jax.ShapeDtypeStruct((NT,), jnp.float32),
            jax.ShapeDtypeStruct((NG,), jnp.float32),
            jax.ShapeDtypeStruct((NJ,), jnp.float32),
        ],
    )(embed_time_t, embed_genre_t, embed_joint_t, w_time, w_side, w_periodic)


@functools.partial(
    pl.kernel,
    out_type=(
        jax.ShapeDtypeStruct((B,), jnp.float32),
        jax.ShapeDtypeStruct((B,), jnp.float32),
        jax.ShapeDtypeStruct((B,), jnp.float32),
        jax.ShapeDtypeStruct((B,), jnp.float32),
    ),
    mesh=plsc.VectorSubcoreMesh(core_axis_name="c", subcore_axis_name="s"),
    compiler_params=pltpu.CompilerParams(
        needs_layout_passes=False, use_tc_tiling_on_sc=True),
    scratch_types=[
        pltpu.VMEM((BW,), jnp.int32),
        pltpu.VMEM((BW,), jnp.int32),
        pltpu.VMEM((BW,), jnp.int32),
        pltpu.VMEM((BW * GEN,), jnp.int32),
        pltpu.VMEM((BW * PLEN,), jnp.float32),
        pltpu.VMEM((BW,), jnp.float32),
        pltpu.VMEM((P_TOTAL,), jnp.float32),
        pltpu.VMEM((BW,), jnp.float32),
        pltpu.VMEM((BW,), jnp.float32),
        pltpu.VMEM((BW,), jnp.float32),
        pltpu.VMEM((BW,), jnp.float32),
        pltpu.SemaphoreType.DMA,
    ],
)
def _sc_kernel(item_h, tr_h, t_h, genre_h, pop_h, pitem_h, pall_h,
               pop_o, time_o, side_o, per_o,
               item_v, tr_v, t_v, genre_v, pop_v, sitem_v, pall_v,
               pop_b, time_b, side_b, per_b, sem):
    wid = lax.axis_index("s") * NC + lax.axis_index("c")
    base = wid * BW

    pltpu.sync_copy(item_h.at[pl.ds(base, BW)], item_v)
    pltpu.sync_copy(tr_h.at[pl.ds(base, BW)], tr_v)
    pltpu.sync_copy(t_h.at[pl.ds(base, BW)], t_v)
    pltpu.sync_copy(genre_h.at[pl.ds(base * GEN, BW * GEN)], genre_v)
    pltpu.sync_copy(pop_h.at[pl.ds(base * PLEN, BW * PLEN)], pop_v)
    pltpu.sync_copy(pall_h, pall_v)

    lanes = lax.iota(jnp.int32, 16)
    zero16 = jnp.zeros((16,), jnp.float32)
    bt = pall_v[pl.ds(OFF_BT, 16)]
    bs = pall_v[pl.ds(OFF_BS, 16)]
    bp = pall_v[pl.ds(OFF_BP, 16)]

    pltpu.async_copy(pitem_h.at[item_v], sitem_v, sem).wait()

    for g in range(BW // LANES):
        off = g * LANES
        sl = pl.ds(off, LANES)
        rowv = lanes + off
        s_item = sitem_v[sl]

        tr16 = tr_v[sl]
        t16 = t_v[sl]
        s_tre = plsc.load_gather(pall_v, [tr16 + OFF_TA])
        s_te = plsc.load_gather(pall_v, [t16 + OFF_TB])
        tmod = lax.rem(t16, NUM_PERIOD)

        sg = zero16
        sj = zero16
        gbase = rowv * GEN
        for j in range(GEN):
            gv = plsc.load_gather(genre_v, [gbase + j])
            sg = sg + plsc.load_gather(pall_v, [gv + OFF_G])
            jid = (gv * NUM_PERIOD + tmod) * jnp.minimum(gv, 1)
            sj = sj + plsc.load_gather(pall_v, [jid + OFF_J])

        popc = plsc.load_gather(pop_v, [rowv * PLEN])

        pop_b[sl] = popc
        time_b[sl] = jnp.maximum(s_tre + s_te + s_item + bt, 0.0)
        side_b[sl] = jnp.maximum(sg * (1.0 / GEN) + bs, 0.0)
        per_b[sl] = jnp.maximum(sj * (1.0 / GEN) + bp, 0.0)

    pltpu.sync_copy(pop_b, pop_o.at[pl.ds(base, BW)])
    pltpu.sync_copy(time_b, time_o.at[pl.ds(base, BW)])
    pltpu.sync_copy(side_b, side_o.at[pl.ds(base, BW)])
    pltpu.sync_copy(per_b, per_o.at[pl.ds(base, BW)])


def kernel(item, time_release, item_genre, item_director, item_actor, time,
           pop_history, pop_gt, valid_pop_len,
           embed_item, embed_time, embed_genre, embed_joint,
           w_periodic, b_periodic, w_time, b_time, w_side, b_side, attn_w):
    pta, ptb, pg, pj = _project(embed_time.T, embed_genre.T, embed_joint.T,
                                w_time, w_side, w_periodic)
    pitem = _project_item(embed_item.T, w_time[:, EMB:2 * EMB])
    pall = jnp.concatenate([
        pta, ptb, pg, pj,
        jnp.zeros((OFF_W2 - OFF_J - NJ,), jnp.float32),
        w_time[0, EMB:2 * EMB],
        jnp.full((16,), b_time[0], jnp.float32),
        jnp.full((16,), b_side[0], jnp.float32),
        jnp.full((16,), b_periodic[0], jnp.float32),
    ])
    pop_o, time_o, side_o, per_o = _sc_kernel(
        item, time_release, time,
        item_genre.reshape(-1), pop_history.reshape(-1),
        pitem, pall)
    w1 = attn_w[1]
    out = time_o * w1 / w1
    return (pop_o[:, None], time_o[:, None], side_o[:, None], per_o[:, None], out)
